# baseline (speedup 1.0000x reference)
import functools

import jax
import jax.numpy as jnp
from jax import lax
from jax.experimental import pallas as pl
from jax.experimental.pallas import tpu as pltpu
from jax.experimental.pallas import tpu_sc as plsc

L = 16

TAI_QMAX = 2047
LOG_QMAX = 4095
TAI_SHIFT = 16
FLUSH = 4
LOW_MASK = (1 << TAI_SHIFT) - 1
KEY_MASK = 0xFFF


def _tc_pack_call(B, W):
  wq = W // 4
  rows_blk = 256

  def body(x_ref, o_ref):
    a = x_ref[:, pl.ds(0, wq)]
    b = x_ref[:, pl.ds(wq, wq)]
    c = x_ref[:, pl.ds(2 * wq, wq)]
    d = x_ref[:, pl.ds(3 * wq, wq)]
    lo = a | (b << 6)
    hi = c | (d << 6)
    o_ref[...] = lo | (hi << 16)

  return pl.pallas_call(
      body,
      grid=(B // rows_blk,),
      in_specs=[pl.BlockSpec((rows_blk, W), lambda i: (i, 0))],
      out_specs=pl.BlockSpec((rows_blk, wq), lambda i: (i, 0)),
      out_shape=jax.ShapeDtypeStruct((B, wq), jnp.int32),
  )


def _make_sc_call(B, W4, num_workers):
  rows_per = B // num_workers
  ch = L
  nchunks = rows_per // ch
  ws = W4 + 17
  mesh = plsc.VectorSubcoreMesh(core_axis_name="c", subcore_axis_name="s")
  info = plsc.get_sparse_core_info()
  nc = info.num_cores

  @functools.partial(
      pl.kernel,
      mesh=mesh,
      compiler_params=pltpu.CompilerParams(needs_layout_passes=False),
      out_type=[
          jax.ShapeDtypeStruct((B,), jnp.float32),
          jax.ShapeDtypeStruct((B,), jnp.float32),
      ],
      scratch_types=[
          pltpu.VMEM((64 * 64,), jnp.int32),
          pltpu.VMEM((L,), jnp.float32),
          pltpu.VMEM((ch, ws), jnp.int32),
          pltpu.VMEM((ch, ws), jnp.int32),
          pltpu.VMEM((rows_per,), jnp.float32),
          pltpu.VMEM((rows_per,), jnp.float32),
          pltpu.SemaphoreType.DMA,
          pltpu.SemaphoreType.DMA,
      ],
  )
  def sc_kernel(idx_hbm, lut_hbm, par_hbm, tai_hbm, cai_hbm,
                lut_v, par_v, buf0, buf1, ot_v, oc_v, sem0, sem1):
    wid = lax.axis_index("s") * nc + lax.axis_index("c")
    base = wid * rows_per
    pltpu.sync_copy(lut_hbm, lut_v)
    pltpu.sync_copy(par_hbm, par_v)
    bufs = (buf0, buf1)
    sems = (sem0, sem1)

    pv = par_v[pl.ds(0, L)]
    c_tai = pv[0]
    c_log = pv[1]
    row_iota = lax.iota(jnp.int32, L)

    def start_chunk(c, slot):
      for r in range(ch):
        pltpu.async_copy(idx_hbm.at[base + c * ch + r],
                         bufs[slot].at[r, pl.ds(0, W4)], sems[slot])

    def wait_chunk(c, slot):
      for r in range(ch):
        pltpu.make_async_copy(idx_hbm.at[base + c * ch + r],
                              bufs[slot].at[r, pl.ds(0, W4)],
                              sems[slot]).wait()

    start_chunk(0, 0)

    def pair_body(c2, _):
      for slot in range(2):
        c = c2 * 2 + slot
        wait_chunk(c, slot)

        @pl.when(c + 1 < nchunks)
        def _(c=c, slot=slot):
          start_chunk(c + 1, 1 - slot)

        zero = jnp.zeros((L,), jnp.int32)
        buf = bufs[slot]

        @plsc.parallel_loop(0, W4 // FLUSH, unroll=4, carry=(zero, zero))
        def accs(j, acc, buf=buf):
          acc_t, acc_q = acc
          col0 = j * FLUSH
          raw = jnp.zeros((L,), jnp.int32)
          for u in range(FLUSH):
            colv = jnp.full((L,), col0, jnp.int32) | u
            w = plsc.load_gather(buf, [row_iota, colv])
            raw = raw + plsc.load_gather(lut_v, [w & jnp.int32(KEY_MASK)])
            raw = raw + plsc.load_gather(
                lut_v, [lax.shift_right_logical(w, 16) & jnp.int32(KEY_MASK)])
          acc_t = acc_t + lax.shift_right_logical(raw, TAI_SHIFT)
          acc_q = acc_q + lax.bitwise_and(raw, jnp.int32(LOW_MASK))
          return (acc_t, acc_q)

        acc_t, acc_q = accs
        ot_v[pl.ds(c * ch, L)] = acc_t.astype(jnp.float32) * c_tai
        oc_v[pl.ds(c * ch, L)] = jnp.exp(acc_q.astype(jnp.float32) * c_log)
      return 0

    lax.fori_loop(0, nchunks // 2, pair_body, 0)
    pltpu.sync_copy(ot_v, tai_hbm.at[pl.ds(base, rows_per)])
    pltpu.sync_copy(oc_v, cai_hbm.at[pl.ds(base, rows_per)])

  return sc_kernel


def kernel(codon_indices, usage_freqs, tai_weights):
  B, W = codon_indices.shape
  info = plsc.get_sparse_core_info()
  num_workers = info.num_cores * info.num_subcores

  neg_log = -jnp.log(usage_freqs / jnp.max(usage_freqs) + 1e-8)
  neg_log = jnp.maximum(neg_log, 0.0)
  qscale = LOG_QMAX / jnp.maximum(jnp.max(neg_log), 1e-30)
  qlog = jnp.clip(jnp.round(neg_log * qscale), 0, LOG_QMAX).astype(jnp.int32)
  tmax = jnp.maximum(jnp.max(tai_weights), 1e-30)
  qtai = jnp.clip(jnp.round(tai_weights * (TAI_QMAX / tmax)), 0,
                  TAI_QMAX).astype(jnp.int32)
  packed = jnp.bitwise_or(jnp.left_shift(qtai, TAI_SHIFT), qlog)
  pair_lut = (packed[None, :] + packed[:, None]).reshape(-1)

  c_tai = tmax / (TAI_QMAX * float(W))
  c_log = -1.0 / (qscale * float(W))
  params = jnp.zeros((L,), jnp.float32).at[0].set(c_tai).at[1].set(c_log)

  nslices = 2
  bs = B // nslices
  tc_call = _tc_pack_call(bs, W)
  sc_call = _make_sc_call(bs, W // 4, num_workers)
  tais, cais = [], []
  for s in range(nslices):
    packed_words = tc_call(
        lax.slice_in_dim(codon_indices, s * bs, (s + 1) * bs))
    tai_s, cai_s = sc_call(packed_words, pair_lut, params)
    tais.append(tai_s)
    cais.append(cai_s)
  return jnp.concatenate(tais), jnp.concatenate(cais)


# speedup vs baseline: 1.2978x; 1.2978x over previous
import functools

import jax
import jax.numpy as jnp
from jax import lax
from jax.experimental import pallas as pl
from jax.experimental.pallas import tpu as pltpu
from jax.experimental.pallas import tpu_sc as plsc

L = 16

TAI_QMAX = 2047
LOG_QMAX = 4095
TAI_SHIFT = 16
FLUSH = 4
LOW_MASK = (1 << TAI_SHIFT) - 1
KEY_MASK = 0xFFF


def _tc_pack_call(B, W):
  wq = W // 4
  rows_blk = 256

  def body(x_ref, o_ref):
    a = x_ref[:, pl.ds(0, wq)]
    b = x_ref[:, pl.ds(wq, wq)]
    c = x_ref[:, pl.ds(2 * wq, wq)]
    d = x_ref[:, pl.ds(3 * wq, wq)]
    lo = a | (b << 6)
    hi = c | (d << 6)
    o_ref[...] = lo | (hi << 16)

  return pl.pallas_call(
      body,
      grid=(B // rows_blk,),
      in_specs=[pl.BlockSpec((rows_blk, W), lambda i: (i, 0))],
      out_specs=pl.BlockSpec((rows_blk, wq), lambda i: (i, 0)),
      out_shape=jax.ShapeDtypeStruct((B, wq), jnp.int32),
  )


def _make_sc_call(B, W4, num_workers):
  rows_per = B // num_workers
  ch = L
  nchunks = rows_per // ch
  ws = W4
  mesh = plsc.VectorSubcoreMesh(core_axis_name="c", subcore_axis_name="s")
  info = plsc.get_sparse_core_info()
  nc = info.num_cores

  @functools.partial(
      pl.kernel,
      mesh=mesh,
      compiler_params=pltpu.CompilerParams(needs_layout_passes=False),
      out_type=[
          jax.ShapeDtypeStruct((B,), jnp.float32),
          jax.ShapeDtypeStruct((B,), jnp.float32),
      ],
      scratch_types=[
          pltpu.VMEM((64 * 64,), jnp.int32),
          pltpu.VMEM((L,), jnp.float32),
          pltpu.VMEM((ch, ws), jnp.int32),
          pltpu.VMEM((ch, ws), jnp.int32),
          pltpu.VMEM((rows_per,), jnp.float32),
          pltpu.VMEM((rows_per,), jnp.float32),
          pltpu.SemaphoreType.DMA,
          pltpu.SemaphoreType.DMA,
      ],
  )
  def sc_kernel(idx_hbm, lut_hbm, par_hbm, tai_hbm, cai_hbm,
                lut_v, par_v, buf0, buf1, ot_v, oc_v, sem0, sem1):
    wid = lax.axis_index("s") * nc + lax.axis_index("c")
    base = wid * rows_per
    pltpu.sync_copy(lut_hbm, lut_v)
    pltpu.sync_copy(par_hbm, par_v)
    bufs = (buf0, buf1)
    sems = (sem0, sem1)

    pv = par_v[pl.ds(0, L)]
    c_tai = pv[0]
    c_log = pv[1]
    row_iota = lax.iota(jnp.int32, L)

    def start_chunk(c, slot):
      pltpu.async_copy(idx_hbm.at[pl.ds(base + c * ch, ch)], bufs[slot],
                       sems[slot])

    def wait_chunk(c, slot):
      pltpu.make_async_copy(idx_hbm.at[pl.ds(base + c * ch, ch)], bufs[slot],
                            sems[slot]).wait()

    start_chunk(0, 0)

    def pair_body(c2, _):
      for slot in range(2):
        c = c2 * 2 + slot
        wait_chunk(c, slot)

        @pl.when(c + 1 < nchunks)
        def _(c=c, slot=slot):
          start_chunk(c + 1, 1 - slot)

        zero = jnp.zeros((L,), jnp.int32)
        buf = bufs[slot]

        @plsc.parallel_loop(0, W4 // FLUSH, unroll=4, carry=(zero, zero))
        def accs(j, acc, buf=buf):
          acc_t, acc_q = acc
          col0 = j * FLUSH
          raw = jnp.zeros((L,), jnp.int32)
          for u in range(FLUSH):
            colv = jnp.full((L,), col0, jnp.int32) | u
            w = plsc.load_gather(buf, [row_iota, colv])
            raw = raw + plsc.load_gather(lut_v, [w & jnp.int32(KEY_MASK)])
            raw = raw + plsc.load_gather(
                lut_v, [lax.shift_right_logical(w, 16) & jnp.int32(KEY_MASK)])
          acc_t = acc_t + lax.shift_right_logical(raw, TAI_SHIFT)
          acc_q = acc_q + lax.bitwise_and(raw, jnp.int32(LOW_MASK))
          return (acc_t, acc_q)

        acc_t, acc_q = accs
        ot_v[pl.ds(c * ch, L)] = acc_t.astype(jnp.float32) * c_tai
        oc_v[pl.ds(c * ch, L)] = jnp.exp(acc_q.astype(jnp.float32) * c_log)
      return 0

    lax.fori_loop(0, nchunks // 2, pair_body, 0)
    pltpu.sync_copy(ot_v, tai_hbm.at[pl.ds(base, rows_per)])
    pltpu.sync_copy(oc_v, cai_hbm.at[pl.ds(base, rows_per)])

  return sc_kernel


def kernel(codon_indices, usage_freqs, tai_weights):
  B, W = codon_indices.shape
  info = plsc.get_sparse_core_info()
  num_workers = info.num_cores * info.num_subcores

  neg_log = -jnp.log(usage_freqs / jnp.max(usage_freqs) + 1e-8)
  neg_log = jnp.maximum(neg_log, 0.0)
  qscale = LOG_QMAX / jnp.maximum(jnp.max(neg_log), 1e-30)
  qlog = jnp.clip(jnp.round(neg_log * qscale), 0, LOG_QMAX).astype(jnp.int32)
  tmax = jnp.maximum(jnp.max(tai_weights), 1e-30)
  qtai = jnp.clip(jnp.round(tai_weights * (TAI_QMAX / tmax)), 0,
                  TAI_QMAX).astype(jnp.int32)
  packed = jnp.bitwise_or(jnp.left_shift(qtai, TAI_SHIFT), qlog)
  pair_lut = (packed[None, :] + packed[:, None]).reshape(-1)

  c_tai = tmax / (TAI_QMAX * float(W))
  c_log = -1.0 / (qscale * float(W))
  params = jnp.zeros((L,), jnp.float32).at[0].set(c_tai).at[1].set(c_log)

  packed_words = _tc_pack_call(B, W)(codon_indices)
  sc_call = _make_sc_call(B, W // 4, num_workers)
  tai, cai = sc_call(packed_words, pair_lut, params)
  return tai, cai
